# baseline (speedup 1.0000x reference)
import jax
import jax.numpy as jnp
from jax import lax
from jax.experimental import pallas as pl
from jax.experimental.pallas import tpu as pltpu
from jax.experimental.pallas import tpu_sc as plsc

_VOCAB = 100000
_B = 4096
_M = 100

_info = plsc.get_sparse_core_info()
_NC, _NS, _L = _info.num_cores, _info.num_subcores, _info.num_lanes
_NW = _NC * _NS
_ROWS = _B // _NW
_GROUPS = _ROWS // _L


def _sc_body(idx_hbm, tab_hbm, out_hbm, idx_v, tab_v, out_v, sem_t, sem_i):
    wid = lax.axis_index("s") * _NC + lax.axis_index("c")
    base = wid * _ROWS

    cp_tab = pltpu.async_copy(tab_hbm, tab_v, sem_t)
    cp_idx = pltpu.async_copy(idx_hbm.at[:, pl.ds(base, _ROWS)], idx_v, sem_i)
    cp_tab.wait()
    cp_idx.wait()

    zeros = tuple(jnp.zeros((_L,), jnp.float32) for _ in range(_GROUPS))

    @plsc.parallel_loop(0, _M, unroll=4, carry=zeros)
    def accs(m, accs_in):
        out = []
        for r in range(_GROUPS):
            ids = idx_v[m, pl.ds(r * _L, _L)]
            vals = plsc.load_gather(tab_v, [ids])
            out.append(accs_in[r] + jnp.where(ids < _VOCAB, vals,
                                              jnp.float32(0.0)))
        return tuple(out)
    for r in range(_GROUPS):
        out_v[pl.ds(r * _L, _L)] = accs[r]

    pltpu.sync_copy(out_v, out_hbm.at[pl.ds(base, _ROWS)])


@jax.jit
def _sc_call(idx_t, tab):
    mesh = plsc.VectorSubcoreMesh(core_axis_name="c", subcore_axis_name="s")
    return pl.kernel(
        _sc_body,
        mesh=mesh,
        out_type=jax.ShapeDtypeStruct((_B,), jnp.float32),
        compiler_params=pltpu.CompilerParams(needs_layout_passes=False),
        scratch_types=[
            pltpu.VMEM((_M, _ROWS), jnp.int32),
            pltpu.VMEM((_VOCAB + 1,), jnp.float32),
            pltpu.VMEM((_ROWS,), jnp.float32),
            pltpu.SemaphoreType.DMA,
            pltpu.SemaphoreType.DMA,
        ],
    )(idx_t, tab)


def kernel(inputs, table):
    return _sc_call(inputs.T, table.reshape(-1))[:, None]


# speedup vs baseline: 1.2328x; 1.2328x over previous
import jax
import jax.numpy as jnp
from jax import lax
from jax.experimental import pallas as pl
from jax.experimental.pallas import tpu as pltpu
from jax.experimental.pallas import tpu_sc as plsc

_VOCAB = 100000
_B = 4096
_M = 100

_info = plsc.get_sparse_core_info()
_NC, _NS, _L = _info.num_cores, _info.num_subcores, _info.num_lanes
_NW = _NC * _NS
_ROWS = _B // _NW
_GROUPS = _ROWS // _L


def _sc_body(idx_hbm, tab_hbm, out_hbm, idx_v, tab_v, tab_sh, out_v,
             sem_t, sem_i):
    sid = lax.axis_index("s")
    wid = sid * _NC + lax.axis_index("c")
    base = wid * _ROWS

    cp_idx = pltpu.async_copy(idx_hbm.at[:, pl.ds(base, _ROWS)], idx_v, sem_i)

    @pl.when(sid == 0)
    def _():
        pltpu.sync_copy(tab_hbm, tab_sh)

    plsc.subcore_barrier()
    cp_tab = pltpu.async_copy(tab_sh, tab_v, sem_t)
    cp_tab.wait()
    cp_idx.wait()

    zeros = tuple(jnp.zeros((_L,), jnp.float32) for _ in range(_GROUPS))

    @plsc.parallel_loop(0, _M, unroll=4, carry=zeros)
    def accs(m, accs_in):
        out = []
        for r in range(_GROUPS):
            ids = idx_v[m, pl.ds(r * _L, _L)]
            vals = plsc.load_gather(tab_v, [ids])
            out.append(accs_in[r] + jnp.where(ids < _VOCAB, vals,
                                              jnp.float32(0.0)))
        return tuple(out)
    for r in range(_GROUPS):
        out_v[pl.ds(r * _L, _L)] = accs[r]

    pltpu.sync_copy(out_v, out_hbm.at[pl.ds(base, _ROWS)])


@jax.jit
def _sc_call(idx_t, tab):
    mesh = plsc.VectorSubcoreMesh(core_axis_name="c", subcore_axis_name="s")
    return pl.kernel(
        _sc_body,
        mesh=mesh,
        out_type=jax.ShapeDtypeStruct((_B,), jnp.float32),
        compiler_params=pltpu.CompilerParams(needs_layout_passes=False),
        scratch_types=[
            pltpu.VMEM((_M, _ROWS), jnp.int32),
            pltpu.VMEM((_VOCAB + 1,), jnp.float32),
            pltpu.VMEM_SHARED((_VOCAB + 1,), jnp.float32),
            pltpu.VMEM((_ROWS,), jnp.float32),
            pltpu.SemaphoreType.DMA,
            pltpu.SemaphoreType.DMA,
        ],
    )(idx_t, tab)


def kernel(inputs, table):
    return _sc_call(inputs.T, table.reshape(-1))[:, None]


# speedup vs baseline: 1.2371x; 1.0035x over previous
import jax
import jax.numpy as jnp
from jax import lax
from jax.experimental import pallas as pl
from jax.experimental.pallas import tpu as pltpu
from jax.experimental.pallas import tpu_sc as plsc

_VOCAB = 100000
_B = 4096
_M = 100
_TAB_PAD = 102400

_info = plsc.get_sparse_core_info()
_NC, _NS, _L = _info.num_cores, _info.num_subcores, _info.num_lanes
_NW = _NC * _NS
_ROWS = _B // _NW
_GROUPS = _ROWS // _L


def _sc_body(idx_hbm, tab_hbm, out_hbm, idx_v, tab_v, tab_sh, out_v,
             sem_t, sem_i):
    sid = lax.axis_index("s")
    wid = sid * _NC + lax.axis_index("c")
    base = wid * _ROWS

    cp_idx = pltpu.async_copy(idx_hbm.at[:, pl.ds(base, _ROWS)], idx_v, sem_i)

    @pl.when(sid == 0)
    def _():
        pltpu.sync_copy(tab_hbm, tab_sh)

    plsc.subcore_barrier()
    cp_tab = pltpu.async_copy(tab_sh, tab_v, sem_t)
    cp_tab.wait()
    cp_idx.wait()

    zeros = tuple(jnp.zeros((_L,), jnp.float32) for _ in range(_GROUPS))

    @plsc.parallel_loop(0, _M, unroll=4, carry=zeros)
    def accs(m, accs_in):
        out = []
        for r in range(_GROUPS):
            ids = idx_v[m, pl.ds(r * _L, _L)]
            vals = plsc.load_gather(tab_v, [ids])
            out.append(accs_in[r] + jnp.where(ids < _VOCAB, vals,
                                              jnp.float32(0.0)))
        return tuple(out)
    for r in range(_GROUPS):
        out_v[pl.ds(r * _L, _L)] = accs[r]

    pltpu.sync_copy(out_v, out_hbm.at[pl.ds(base, _ROWS)])


@jax.jit
def _sc_call(idx_t, tab):
    mesh = plsc.VectorSubcoreMesh(core_axis_name="c", subcore_axis_name="s")
    return pl.kernel(
        _sc_body,
        mesh=mesh,
        out_type=jax.ShapeDtypeStruct((_B,), jnp.float32),
        compiler_params=pltpu.CompilerParams(needs_layout_passes=False),
        scratch_types=[
            pltpu.VMEM((_M, _ROWS), jnp.int32),
            pltpu.VMEM((_TAB_PAD,), jnp.float32),
            pltpu.VMEM_SHARED((_TAB_PAD,), jnp.float32),
            pltpu.VMEM((_ROWS,), jnp.float32),
            pltpu.SemaphoreType.DMA,
            pltpu.SemaphoreType.DMA,
        ],
    )(idx_t, tab)


def kernel(inputs, table):
    tab = jnp.pad(table, ((0, _TAB_PAD - (_VOCAB + 1)), (0, 0)))
    return _sc_call(inputs.T, tab.reshape(-1))[:, None]
